# Initial kernel scaffold; baseline (speedup 1.0000x reference)
import jax
import jax.numpy as jnp
from jax.experimental import pallas as pl


def kernel(idx, targets, table):
    raise NotImplementedError("write your pallas kernel here")


# speedup vs baseline: 1.9111x; 1.9111x over previous
import jax
import jax.numpy as jnp
from jax import lax
from jax.experimental import pallas as pl
from jax.experimental.pallas import tpu as pltpu
from jax.experimental.pallas import tpu_sc as plsc

VOCAB = 8192
BB, TT = 8, 2048
N = BB * TT
NC, NS, L = 2, 16, 16
NW = NC * NS
RPW = N // NW
CHUNK = 4
NCH = RPW // CHUNK

_mesh = plsc.VectorSubcoreMesh(core_axis_name="c", subcore_axis_name="s")


def _gather_body(table_hbm, idx_hbm, out_hbm, idx_v, buf0, buf1, sem0, sem1):
    wid = lax.axis_index("s") * NC + lax.axis_index("c")
    base = wid * RPW
    pltpu.sync_copy(idx_hbm.at[wid], idx_v)

    bufs = (buf0, buf1)
    sems = (sem0, sem1)

    pltpu.async_copy(table_hbm.at[idx_v.at[0]], buf0, sem0)

    def body(i, _):
        for b in range(2):
            ib = 2 * i + b
            nxt = ib + 1
            @pl.when(nxt < NCH)
            def _():
                pltpu.async_copy(table_hbm.at[idx_v.at[nxt]],
                                 bufs[1 - b], sems[1 - b])
            pltpu.make_async_copy(table_hbm.at[idx_v.at[ib]],
                                  bufs[b], sems[b]).wait()
            pltpu.sync_copy(bufs[b], out_hbm.at[pl.ds(base + ib * CHUNK, CHUNK)])
        return 0

    lax.fori_loop(0, NCH // 2, body, 0)


def _sc_gather(table, idx3):
    k = pl.kernel(
        _gather_body,
        out_type=jax.ShapeDtypeStruct((N, VOCAB), jnp.float32),
        mesh=_mesh,
        scratch_types=[
            pltpu.VMEM((NCH, CHUNK), jnp.int32),
            pltpu.VMEM((CHUNK, VOCAB), jnp.float32),
            pltpu.VMEM((CHUNK, VOCAB), jnp.float32),
            pltpu.SemaphoreType.DMA,
            pltpu.SemaphoreType.DMA,
        ],
    )
    return k(table, idx3)


_LSE_R = 128


def _lse_kernel(tab_ref, out_ref):
    x = tab_ref[...]
    m = jnp.max(x, axis=1, keepdims=True)
    s = jnp.sum(jnp.exp(x - m), axis=1, keepdims=True)
    out_ref[...] = m + jnp.log(s)


def _tc_lse(table):
    out = pl.pallas_call(
        _lse_kernel,
        grid=(VOCAB // _LSE_R,),
        in_specs=[pl.BlockSpec((_LSE_R, VOCAB), lambda i: (i, 0))],
        out_specs=pl.BlockSpec((_LSE_R, 1), lambda i: (i, 0)),
        out_shape=jax.ShapeDtypeStruct((VOCAB, 1), jnp.float32),
    )(table)
    return out.reshape(VOCAB)


def _loss_body(tabflat_hbm, lse_hbm, idx_hbm, tgt_hbm, out_hbm,
               idx_v, tgt_v, fidx_v, lse_g, picked, acc_v, sem):
    wid = lax.axis_index("s") * NC + lax.axis_index("c")
    base = wid * RPW
    pltpu.sync_copy(idx_hbm.at[pl.ds(base, RPW)], idx_v)
    pltpu.sync_copy(tgt_hbm.at[pl.ds(base, RPW)], tgt_v)

    def bidx(j, _):
        off = j * L
        fidx_v[pl.ds(off, L)] = idx_v[pl.ds(off, L)] * VOCAB + tgt_v[pl.ds(off, L)]
        return 0
    lax.fori_loop(0, RPW // L, bidx, 0)

    def gather128(j, _):
        off = j * 128
        pltpu.async_copy(tabflat_hbm.at[fidx_v.at[pl.ds(off, 128)]],
                         picked.at[pl.ds(off, 128)], sem).wait()
        pltpu.async_copy(lse_hbm.at[idx_v.at[pl.ds(off, 128)]],
                         lse_g.at[pl.ds(off, 128)], sem).wait()
        return 0
    lax.fori_loop(0, RPW // 128, gather128, 0)

    def accum(j, acc):
        off = j * L
        return acc + (lse_g[pl.ds(off, L)] - picked[pl.ds(off, L)])
    acc = lax.fori_loop(0, RPW // L, accum, jnp.zeros((L,), jnp.float32))
    acc_v[...] = acc
    pltpu.sync_copy(acc_v, out_hbm.at[wid])


def _sc_loss_partials(table_flat, lse, idx_flat, tgt_flat):
    k = pl.kernel(
        _loss_body,
        out_type=jax.ShapeDtypeStruct((NW, L), jnp.float32),
        mesh=_mesh,
        scratch_types=[
            pltpu.VMEM((RPW,), jnp.int32),
            pltpu.VMEM((RPW,), jnp.int32),
            pltpu.VMEM((RPW,), jnp.int32),
            pltpu.VMEM((RPW,), jnp.float32),
            pltpu.VMEM((RPW,), jnp.float32),
            pltpu.VMEM((L,), jnp.float32),
            pltpu.SemaphoreType.DMA,
        ],
    )
    return k(table_flat, lse, idx_flat, tgt_flat)


def kernel(idx, targets, table):
    idx_flat = idx.reshape(N).astype(jnp.int32)
    tgt_flat = targets.reshape(N).astype(jnp.int32)
    table_flat = table.reshape(VOCAB * VOCAB)

    logits_flat = _sc_gather(table, idx_flat.reshape(NW, NCH, CHUNK))
    lse = _tc_lse(table)
    partials = _sc_loss_partials(table_flat, lse, idx_flat, tgt_flat)

    loss = jnp.sum(partials) / jnp.float32(N)
    return logits_flat.reshape(BB, TT, VOCAB), loss


# speedup vs baseline: 2.6260x; 1.3741x over previous
import jax
import jax.numpy as jnp
from jax import lax
from jax.experimental import pallas as pl
from jax.experimental.pallas import tpu as pltpu
from jax.experimental.pallas import tpu_sc as plsc

VOCAB = 8192
BB, TT = 8, 2048
N = BB * TT
NC, NS, L = 2, 16, 16
NW = NC * NS
RPW = N // NW
CHUNK = 4
NCH = RPW // CHUNK

_mesh = plsc.VectorSubcoreMesh(core_axis_name="c", subcore_axis_name="s")


def _gather_body(table_hbm, idx_hbm, tgt_hbm, out_hbm, psum_hbm,
                 idx_v, tgt_v, psum_v, buf0, buf1, sem0, sem1):
    wid = lax.axis_index("s") * NC + lax.axis_index("c")
    base = wid * RPW
    pltpu.sync_copy(idx_hbm.at[wid], idx_v)
    pltpu.sync_copy(tgt_hbm.at[wid], tgt_v.at[pl.ds(0, RPW)])

    bufs = (buf0, buf1)
    sems = (sem0, sem1)
    lanes = lax.iota(jnp.int32, L)

    pltpu.async_copy(table_hbm.at[idx_v.at[0]], buf0, sem0)

    def body(i, psum):
        cols16 = tgt_v[pl.ds(i * 2 * CHUNK, L)]
        for b in range(2):
            ib = 2 * i + b
            nxt = ib + 1
            @pl.when(nxt < NCH)
            def _():
                pltpu.async_copy(table_hbm.at[idx_v.at[nxt]],
                                 bufs[1 - b], sems[1 - b])
            pltpu.make_async_copy(table_hbm.at[idx_v.at[ib]],
                                  bufs[b], sems[b]).wait()
            for r in range(CHUNK):
                c = cols16[b * CHUNK + r]
                j0 = pl.multiple_of((c >> 4) << 4, L)
                l0 = c & (L - 1)
                v = bufs[b][r, pl.ds(j0, L)]
                psum = psum + jnp.where(lanes == l0, v, 0.0)
            pltpu.sync_copy(bufs[b], out_hbm.at[pl.ds(base + ib * CHUNK, CHUNK)])
        return psum

    psum = lax.fori_loop(0, NCH // 2, body, jnp.zeros((L,), jnp.float32))
    psum_v[...] = psum
    pltpu.sync_copy(psum_v, psum_hbm.at[wid])


def _sc_gather(table, idx3, tgt2):
    k = pl.kernel(
        _gather_body,
        out_type=(jax.ShapeDtypeStruct((N, VOCAB), jnp.float32),
                  jax.ShapeDtypeStruct((NW, L), jnp.float32)),
        mesh=_mesh,
        scratch_types=[
            pltpu.VMEM((NCH, CHUNK), jnp.int32),
            pltpu.VMEM((RPW + L,), jnp.int32),
            pltpu.VMEM((L,), jnp.float32),
            pltpu.VMEM((CHUNK, VOCAB), jnp.float32),
            pltpu.VMEM((CHUNK, VOCAB), jnp.float32),
            pltpu.SemaphoreType.DMA,
            pltpu.SemaphoreType.DMA,
        ],
    )
    return k(table, idx3, tgt2)


_LSE_R = 128


def _lse_kernel(tab_ref, out_ref):
    x = tab_ref[...]
    m = jnp.max(x, axis=1, keepdims=True)
    s = jnp.sum(jnp.exp(x - m), axis=1, keepdims=True)
    out_ref[...] = m + jnp.log(s)


def _tc_lse(table):
    out = pl.pallas_call(
        _lse_kernel,
        grid=(VOCAB // _LSE_R,),
        in_specs=[pl.BlockSpec((_LSE_R, VOCAB), lambda i: (i, 0))],
        out_specs=pl.BlockSpec((_LSE_R, 1), lambda i: (i, 0)),
        out_shape=jax.ShapeDtypeStruct((VOCAB, 1), jnp.float32),
    )(table)
    return out.reshape(VOCAB)


def _loss_body(lse_hbm, idx_hbm, psum_hbm, out_hbm,
               idx_v, lse_g, ps_v, acc_v, sem):
    wid = lax.axis_index("s") * NC + lax.axis_index("c")
    pltpu.sync_copy(idx_hbm.at[wid], idx_v)
    pltpu.sync_copy(psum_hbm.at[wid], ps_v)

    def gather128(j, _):
        off = j * 128
        pltpu.async_copy(lse_hbm.at[idx_v.at[pl.ds(off, 128)]],
                         lse_g.at[pl.ds(off, 128)], sem).wait()
        return 0
    lax.fori_loop(0, RPW // 128, gather128, 0)

    def accum(j, acc):
        off = j * L
        return acc + lse_g[pl.ds(off, L)]
    acc = lax.fori_loop(0, RPW // L, accum, jnp.zeros((L,), jnp.float32))
    acc_v[...] = acc - ps_v[...]
    pltpu.sync_copy(acc_v, out_hbm.at[wid])


def _sc_loss_partials(lse, idx2, psum):
    k = pl.kernel(
        _loss_body,
        out_type=jax.ShapeDtypeStruct((NW, L), jnp.float32),
        mesh=_mesh,
        scratch_types=[
            pltpu.VMEM((RPW,), jnp.int32),
            pltpu.VMEM((RPW,), jnp.float32),
            pltpu.VMEM((L,), jnp.float32),
            pltpu.VMEM((L,), jnp.float32),
            pltpu.SemaphoreType.DMA,
        ],
    )
    return k(lse, idx2, psum)


def kernel(idx, targets, table):
    idx_flat = idx.reshape(N).astype(jnp.int32)
    tgt_flat = targets.reshape(N).astype(jnp.int32)

    logits_flat, psum = _sc_gather(table, idx_flat.reshape(NW, NCH, CHUNK),
                                   tgt_flat.reshape(NW, RPW))
    lse = _tc_lse(table)
    partials = _sc_loss_partials(lse, idx_flat.reshape(NW, RPW), psum)

    loss = jnp.sum(partials) / jnp.float32(N)
    return logits_flat.reshape(BB, TT, VOCAB), loss
